# baseline (speedup 1.0000x reference)
import functools

import jax
import jax.numpy as jnp
from jax import lax
from jax.experimental import pallas as pl
from jax.experimental.pallas import tpu as pltpu
from jax.experimental.pallas import tpu_sc as plsc

N = 10000
E = 320000
D = 128

NC, NS = 2, 16
NW = NC * NS
B = 128
SHIFT = 14
CH0 = 112
CH1 = 48
C1CHK = NS * CH1
NCHK = NS * (CH0 + CH1)
EPAD = NCHK * B
DCH = NCHK // NW
NACC = 10240
RPT = NACC // NS
ZR = NACC // NS


def _fill_vmem_2d(ref, rows, width, value):
    val = jnp.full((16,), value, jnp.float32)

    def row(i, _):
        for j in range(width // 16):
            ref[i, pl.ds(j * 16, 16)] = val
        return 0

    lax.fori_loop(0, rows, row, 0)


def _zero_acc_slice(zbuf, acc, s):
    zb = s * ZR
    done = 0
    while done < ZR:
        n = min(B, ZR - done)
        pltpu.sync_copy(zbuf.at[pl.ds(0, n)], acc.at[pl.ds(zb + done, n)])
        done += n


def _unpack_dst(comb, i, dstq):
    for j in range(B // 16):
        v = comb[i, pl.ds(j * 16, 16)]
        dstq[pl.ds(j * 16, 16)] = lax.bitwise_and(v, (1 << SHIFT) - 1)


def _unpack_src(comb, i, srcq):
    for j in range(B // 16):
        v = comb[i, pl.ds(j * 16, 16)]
        srcq[pl.ds(j * 16, 16)] = lax.shift_right_logical(v, SHIFT)


@functools.cache
def _sc_mesh():
    return plsc.VectorSubcoreMesh(core_axis_name="c", subcore_axis_name="s",
                                  num_cores=NC, num_subcores=NS)


@functools.cache
def _sc_degree_kernel():
    return pl.kernel(
        _sc_degree_body,
        out_type=jax.ShapeDtypeStruct((NC, NACC, 16), jnp.float32),
        mesh=_sc_mesh(),
        scratch_types=[
            pltpu.VMEM((DCH, B), jnp.int32),
            pltpu.VMEM((B,), jnp.int32),
            pltpu.VMEM((B, 16), jnp.float32),
            pltpu.VMEM((B, 16), jnp.float32),
            pltpu.VMEM_SHARED((NACC, 16), jnp.float32),
        ],
    )


def _sc_degree(comb):
    return _sc_degree_kernel()(comb)


def _sc_degree_body(comb_hbm, out_hbm, comb, dstq, ones, zbuf, dacc):
    c = lax.axis_index("c")
    s = lax.axis_index("s")
    wid = s * NC + c
    _fill_vmem_2d(ones, B, 16, 1.0)
    _fill_vmem_2d(zbuf, B, 16, 0.0)
    _zero_acc_slice(zbuf, dacc, s)
    pltpu.sync_copy(comb_hbm.at[pl.ds(wid * DCH, DCH)], comb)
    plsc.subcore_barrier()

    def chunk(i, _):
        _unpack_dst(comb, i, dstq)
        pltpu.sync_copy(ones, dacc.at[dstq], add=True)
        return 0

    lax.fori_loop(0, DCH, chunk, 0)
    plsc.subcore_barrier()
    ob = s * RPT
    pltpu.sync_copy(dacc.at[pl.ds(ob, RPT)], out_hbm.at[c, pl.ds(ob, RPT)])


@functools.cache
def _sc_prop_kernel():
    return pl.kernel(
        _sc_prop_body,
        out_type=jax.ShapeDtypeStruct((NC, NACC, D), jnp.float32),
        mesh=_sc_mesh(),
        scratch_types=[
            pltpu.VMEM((CH0, B), jnp.int32),
            pltpu.VMEM((B,), jnp.int32),
            pltpu.VMEM((B,), jnp.int32),
            pltpu.VMEM((B,), jnp.int32),
            pltpu.VMEM((B,), jnp.int32),
            pltpu.VMEM((B, D), jnp.float32),
            pltpu.VMEM((B, D), jnp.float32),
            pltpu.VMEM_SHARED((NACC, D), jnp.float32),
            pltpu.SemaphoreType.DMA,
            pltpu.SemaphoreType.DMA,
        ],
    )


def _sc_prop(g, comb):
    return _sc_prop_kernel()(g, comb)


def _prop_chunks(g_hbm, comb_hbm, comb, srcq0, srcq1, dstq0, dstq1,
                 rows0, rows1, acc, sem0, sem1, base, nch):
    pltpu.sync_copy(comb_hbm.at[pl.ds(base, nch)], comb.at[pl.ds(0, nch)])
    _unpack_src(comb, 0, srcq0)
    _unpack_dst(comb, 0, dstq0)
    pltpu.async_copy(g_hbm.at[srcq0], rows0, sem0)
    _unpack_src(comb, 1, srcq1)
    _unpack_dst(comb, 1, dstq1)

    def pair(k, _):
        i = 2 * k
        pltpu.make_async_copy(g_hbm.at[srcq0], rows0, sem0).wait()
        pltpu.async_copy(g_hbm.at[srcq1], rows1, sem1)
        pltpu.sync_copy(rows0, acc.at[dstq0], add=True)
        _unpack_src(comb, i + 2, srcq0)
        _unpack_dst(comb, i + 2, dstq0)
        pltpu.make_async_copy(g_hbm.at[srcq1], rows1, sem1).wait()
        pltpu.async_copy(g_hbm.at[srcq0], rows0, sem0)
        pltpu.sync_copy(rows1, acc.at[dstq1], add=True)
        _unpack_src(comb, i + 3, srcq1)
        _unpack_dst(comb, i + 3, dstq1)
        return 0

    lax.fori_loop(0, nch // 2 - 1, pair, 0)
    pltpu.make_async_copy(g_hbm.at[srcq0], rows0, sem0).wait()
    pltpu.async_copy(g_hbm.at[srcq1], rows1, sem1)
    pltpu.sync_copy(rows0, acc.at[dstq0], add=True)
    pltpu.make_async_copy(g_hbm.at[srcq1], rows1, sem1).wait()
    pltpu.sync_copy(rows1, acc.at[dstq1], add=True)


def _sc_prop_body(g_hbm, comb_hbm, out_hbm, comb, srcq0, srcq1, dstq0, dstq1,
                  rows0, rows1, acc, sem0, sem1):
    c = lax.axis_index("c")
    s = lax.axis_index("s")
    _fill_vmem_2d(rows0, B, D, 0.0)
    _zero_acc_slice(rows0, acc, s)
    plsc.subcore_barrier()

    args = (g_hbm, comb_hbm, comb, srcq0, srcq1, dstq0, dstq1,
            rows0, rows1, acc, sem0, sem1)

    @pl.when(c == 0)
    def _():
        _prop_chunks(*args, C1CHK + s * CH0, CH0)

    @pl.when(c != 0)
    def _():
        _prop_chunks(*args, s * CH1, CH1)

    plsc.subcore_barrier()
    ob = s * RPT
    pltpu.sync_copy(acc.at[pl.ds(ob, RPT)], out_hbm.at[c, pl.ds(ob, RPT)])


RB = 2000
GRID = N // RB


def _rb(width):
    return pl.BlockSpec((RB, width), lambda i: (i, 0))


def _p0(width):
    return pl.BlockSpec((1, RB, width), lambda i: (0, i, 0))


def _p1(width):
    return pl.BlockSpec((1, RB, width), lambda i: (1, i, 0))


def _full(shape):
    return pl.BlockSpec(shape, lambda i: tuple(0 for _ in shape))


def _tc_scale_body(d0, d1, x, g1, dis):
    deg = d0[0, :, :1] + d1[0, :, :1] + 1.0
    dv = lax.rsqrt(deg)
    dis[...] = dv
    g1[...] = x[...] * dv


def _tc_scale(degp, x):
    return pl.pallas_call(
        _tc_scale_body,
        grid=(GRID,),
        in_specs=[_p0(16), _p1(16), _rb(D)],
        out_specs=[_rb(D), _rb(1)],
        out_shape=[
            jax.ShapeDtypeStruct((N, D), jnp.float32),
            jax.ShapeDtypeStruct((N, 1), jnp.float32),
        ],
    )(degp, degp, x)


def _tc_z1_body(s0, s1, g1, dis, W1, b1, z, st):
    p = (s0[0] + s1[0] + g1[...]) * dis[...]
    zv = jnp.dot(p, W1[...], preferred_element_type=jnp.float32) + b1[...]
    z[...] = zv
    cs = jnp.sum(zv, axis=0).reshape(1, 1, -1)
    cq = jnp.sum(zv * zv, axis=0).reshape(1, 1, -1)
    st[...] = jnp.concatenate([cs, cq], axis=1)


def _tc_z1(sp, g1, dis, W1, b1):
    dh = W1.shape[1]
    return pl.pallas_call(
        _tc_z1_body,
        grid=(GRID,),
        in_specs=[_p0(D), _p1(D), _rb(D), _rb(1), _full((D, dh)), _full((1, dh))],
        out_specs=[_rb(dh), pl.BlockSpec((1, 2, dh), lambda i: (i, 0, 0))],
        out_shape=[
            jax.ShapeDtypeStruct((N, dh), jnp.float32),
            jax.ShapeDtypeStruct((GRID, 2, dh), jnp.float32),
        ],
    )(sp, sp, g1, dis, W1, b1)


def _bn_relu(z, st, gamma, beta):
    ssum = jnp.sum(st[...][:, 0, :], axis=0)
    ssq = jnp.sum(st[...][:, 1, :], axis=0)
    mean = ssum * (1.0 / N)
    var = ssq * (1.0 / N) - mean * mean
    inv = lax.rsqrt(var + 1e-5)
    g = gamma[...][0]
    sc = (inv * g).reshape(1, -1)
    sh = (beta[...][0] - mean * inv * g).reshape(1, -1)
    return jnp.maximum(z[...] * sc + sh, 0.0)


def _tc_x1_body(z, st, gamma, beta, dis, W2, Wp, g2, r):
    x1 = _bn_relu(z, st, gamma, beta)
    g2[...] = jnp.dot(x1, W2[...], preferred_element_type=jnp.float32) * dis[...]
    r[...] = jnp.dot(x1, Wp[...], preferred_element_type=jnp.float32)


def _tc_x1(z1, st1, gamma, beta, dis, W2, Wp):
    dh = z1.shape[1]
    return pl.pallas_call(
        _tc_x1_body,
        grid=(GRID,),
        in_specs=[_rb(dh), _full((GRID, 2, dh)), _full((1, dh)), _full((1, dh)),
                  _rb(1), _full((dh, D)), _full((dh, D))],
        out_specs=[_rb(D), _rb(D)],
        out_shape=[
            jax.ShapeDtypeStruct((N, D), jnp.float32),
            jax.ShapeDtypeStruct((N, D), jnp.float32),
        ],
    )(z1, st1, gamma, beta, dis, W2, Wp)


def _tc_z2_body(s0, s1, g2, dis, b2, z, st):
    zv = (s0[0] + s1[0] + g2[...]) * dis[...] + b2[...]
    z[...] = zv
    cs = jnp.sum(zv, axis=0).reshape(1, 1, -1)
    cq = jnp.sum(zv * zv, axis=0).reshape(1, 1, -1)
    st[...] = jnp.concatenate([cs, cq], axis=1)


def _tc_z2(sp, g2, dis, b2):
    return pl.pallas_call(
        _tc_z2_body,
        grid=(GRID,),
        in_specs=[_p0(D), _p1(D), _rb(D), _rb(1), _full((1, D))],
        out_specs=[_rb(D), pl.BlockSpec((1, 2, D), lambda i: (i, 0, 0))],
        out_shape=[
            jax.ShapeDtypeStruct((N, D), jnp.float32),
            jax.ShapeDtypeStruct((GRID, 2, D), jnp.float32),
        ],
    )(sp, sp, g2, dis, b2)


def _tc_x2_body(z, st, gamma, beta, dis, W3, g3):
    x2 = _bn_relu(z, st, gamma, beta)
    g3[...] = jnp.dot(x2, W3[...], preferred_element_type=jnp.float32) * dis[...]


def _tc_x2(z2, st2, gamma, beta, dis, W3):
    return pl.pallas_call(
        _tc_x2_body,
        grid=(GRID,),
        in_specs=[_rb(D), _full((GRID, 2, D)), _full((1, D)), _full((1, D)),
                  _rb(1), _full((D, D))],
        out_specs=_rb(D),
        out_shape=jax.ShapeDtypeStruct((N, D), jnp.float32),
    )(z2, st2, gamma, beta, dis, W3)


def _tc_out_body(s0, s1, g3, dis, b3, r, bp, o):
    o[...] = ((s0[0] + s1[0] + g3[...]) * dis[...] + b3[...] + r[...]
              + bp[...])


def _tc_out(sp, g3, dis, b3, r, bp):
    return pl.pallas_call(
        _tc_out_body,
        grid=(GRID,),
        in_specs=[_p0(D), _p1(D), _rb(D), _rb(1), _full((1, D)), _rb(D),
                  _full((1, D))],
        out_specs=_rb(D),
        out_shape=jax.ShapeDtypeStruct((N, D), jnp.float32),
    )(sp, sp, g3, dis, b3, r, bp)


def kernel(x, edge_index, W1, b1, gamma1, beta1, W2, b2, gamma2, beta2,
           W3, b3, Wproj, bproj):
    pad = EPAD - E
    src = jnp.concatenate([edge_index[0], jnp.zeros((pad,), jnp.int32)])
    dst = jnp.concatenate([edge_index[1], jnp.full((pad,), N, jnp.int32)])
    comb = ((src << SHIFT) | dst).reshape(NCHK, B)

    degp = _sc_degree(comb)
    g1, dis = _tc_scale(degp, x)

    s1 = _sc_prop(g1, comb)
    z1, st1 = _tc_z1(s1, g1, dis, W1, b1.reshape(1, -1))
    g2, r = _tc_x1(z1, st1, gamma1.reshape(1, -1), beta1.reshape(1, -1),
                   dis, W2, Wproj)

    s2 = _sc_prop(g2, comb)
    z2, st2 = _tc_z2(s2, g2, dis, b2.reshape(1, -1))
    g3 = _tc_x2(z2, st2, gamma2.reshape(1, -1), beta2.reshape(1, -1), dis, W3)

    s3 = _sc_prop(g3, comb)
    return _tc_out(s3, g3, dis, b3.reshape(1, -1), r, bproj.reshape(1, -1))


# speedup vs baseline: 1.1064x; 1.1064x over previous
import functools

import jax
import jax.numpy as jnp
from jax import lax
from jax.experimental import pallas as pl
from jax.experimental.pallas import tpu as pltpu
from jax.experimental.pallas import tpu_sc as plsc

N = 10000
E = 320000
D = 128

NC, NS = 2, 16
NW = NC * NS
B = 128
SHIFT = 14
CH0 = 80
CH1 = 80
C1CHK = NS * CH1
NCHK = NS * (CH0 + CH1)
EPAD = NCHK * B
DCH = NCHK // NW
NACC = 10240
RPT = NACC // NS
ZR = NACC // NS


def _fill_vmem_2d(ref, rows, width, value):
    val = jnp.full((16,), value, jnp.float32)

    def row(i, _):
        for j in range(width // 16):
            ref[i, pl.ds(j * 16, 16)] = val
        return 0

    lax.fori_loop(0, rows, row, 0)


def _zero_acc_slice(zbuf, acc, s):
    zb = s * ZR
    done = 0
    while done < ZR:
        n = min(B, ZR - done)
        pltpu.sync_copy(zbuf.at[pl.ds(0, n)], acc.at[pl.ds(zb + done, n)])
        done += n


def _unpack_dst(comb, i, dstq):
    for j in range(B // 16):
        v = comb[i, pl.ds(j * 16, 16)]
        dstq[pl.ds(j * 16, 16)] = lax.bitwise_and(v, (1 << SHIFT) - 1)


def _unpack_src(comb, i, srcq):
    for j in range(B // 16):
        v = comb[i, pl.ds(j * 16, 16)]
        srcq[pl.ds(j * 16, 16)] = lax.shift_right_logical(v, SHIFT)


@functools.cache
def _sc_mesh():
    return plsc.VectorSubcoreMesh(core_axis_name="c", subcore_axis_name="s",
                                  num_cores=NC, num_subcores=NS)


@functools.cache
def _sc_degree_kernel():
    return pl.kernel(
        _sc_degree_body,
        out_type=jax.ShapeDtypeStruct((NC, NACC, 16), jnp.float32),
        mesh=_sc_mesh(),
        scratch_types=[
            pltpu.VMEM((DCH, B), jnp.int32),
            pltpu.VMEM((B,), jnp.int32),
            pltpu.VMEM((B, 16), jnp.float32),
            pltpu.VMEM((B, 16), jnp.float32),
            pltpu.VMEM_SHARED((NACC, 16), jnp.float32),
        ],
    )


def _sc_degree(comb):
    return _sc_degree_kernel()(comb)


def _sc_degree_body(comb_hbm, out_hbm, comb, dstq, ones, zbuf, dacc):
    c = lax.axis_index("c")
    s = lax.axis_index("s")
    wid = s * NC + c
    _fill_vmem_2d(ones, B, 16, 1.0)
    _fill_vmem_2d(zbuf, B, 16, 0.0)
    _zero_acc_slice(zbuf, dacc, s)
    pltpu.sync_copy(comb_hbm.at[pl.ds(wid * DCH, DCH)], comb)
    plsc.subcore_barrier()

    def chunk(i, _):
        _unpack_dst(comb, i, dstq)
        pltpu.sync_copy(ones, dacc.at[dstq], add=True)
        return 0

    lax.fori_loop(0, DCH, chunk, 0)
    plsc.subcore_barrier()
    ob = s * RPT
    pltpu.sync_copy(dacc.at[pl.ds(ob, RPT)], out_hbm.at[c, pl.ds(ob, RPT)])


@functools.cache
def _sc_prop_kernel():
    return pl.kernel(
        _sc_prop_body,
        out_type=jax.ShapeDtypeStruct((NC, NACC, D), jnp.float32),
        mesh=_sc_mesh(),
        scratch_types=[
            pltpu.VMEM((CH0, B), jnp.int32),
            pltpu.VMEM((B,), jnp.int32),
            pltpu.VMEM((B,), jnp.int32),
            pltpu.VMEM((B,), jnp.int32),
            pltpu.VMEM((B,), jnp.int32),
            pltpu.VMEM((B, D), jnp.float32),
            pltpu.VMEM((B, D), jnp.float32),
            pltpu.VMEM_SHARED((NACC, D), jnp.float32),
            pltpu.SemaphoreType.DMA,
            pltpu.SemaphoreType.DMA,
        ],
    )


def _sc_prop(g, comb):
    return _sc_prop_kernel()(g, comb)


def _prop_chunks(g_hbm, comb_hbm, comb, srcq0, srcq1, dstq0, dstq1,
                 rows0, rows1, acc, sem0, sem1, base, nch):
    pltpu.sync_copy(comb_hbm.at[pl.ds(base, nch)], comb.at[pl.ds(0, nch)])
    _unpack_src(comb, 0, srcq0)
    _unpack_dst(comb, 0, dstq0)
    pltpu.async_copy(g_hbm.at[srcq0], rows0, sem0)
    _unpack_src(comb, 1, srcq1)
    _unpack_dst(comb, 1, dstq1)

    def pair(k, _):
        i = 2 * k
        pltpu.make_async_copy(g_hbm.at[srcq0], rows0, sem0).wait()
        pltpu.async_copy(g_hbm.at[srcq1], rows1, sem1)
        pltpu.sync_copy(rows0, acc.at[dstq0], add=True)
        _unpack_src(comb, i + 2, srcq0)
        _unpack_dst(comb, i + 2, dstq0)
        pltpu.make_async_copy(g_hbm.at[srcq1], rows1, sem1).wait()
        pltpu.async_copy(g_hbm.at[srcq0], rows0, sem0)
        pltpu.sync_copy(rows1, acc.at[dstq1], add=True)
        _unpack_src(comb, i + 3, srcq1)
        _unpack_dst(comb, i + 3, dstq1)
        return 0

    lax.fori_loop(0, nch // 2 - 1, pair, 0)
    pltpu.make_async_copy(g_hbm.at[srcq0], rows0, sem0).wait()
    pltpu.async_copy(g_hbm.at[srcq1], rows1, sem1)
    pltpu.sync_copy(rows0, acc.at[dstq0], add=True)
    pltpu.make_async_copy(g_hbm.at[srcq1], rows1, sem1).wait()
    pltpu.sync_copy(rows1, acc.at[dstq1], add=True)


def _sc_prop_body(g_hbm, comb_hbm, out_hbm, comb, srcq0, srcq1, dstq0, dstq1,
                  rows0, rows1, acc, sem0, sem1):
    c = lax.axis_index("c")
    s = lax.axis_index("s")
    _fill_vmem_2d(rows0, B, D, 0.0)
    _zero_acc_slice(rows0, acc, s)
    plsc.subcore_barrier()

    args = (g_hbm, comb_hbm, comb, srcq0, srcq1, dstq0, dstq1,
            rows0, rows1, acc, sem0, sem1)

    @pl.when(c == 0)
    def _():
        _prop_chunks(*args, C1CHK + s * CH0, CH0)

    @pl.when(c != 0)
    def _():
        _prop_chunks(*args, s * CH1, CH1)

    plsc.subcore_barrier()
    ob = s * RPT
    pltpu.sync_copy(acc.at[pl.ds(ob, RPT)], out_hbm.at[c, pl.ds(ob, RPT)])


RB = 2000
GRID = N // RB


def _rb(width):
    return pl.BlockSpec((RB, width), lambda i: (i, 0))


def _p0(width):
    return pl.BlockSpec((1, RB, width), lambda i: (0, i, 0))


def _p1(width):
    return pl.BlockSpec((1, RB, width), lambda i: (1, i, 0))


def _full(shape):
    return pl.BlockSpec(shape, lambda i: tuple(0 for _ in shape))


def _tc_scale_body(d0, d1, x, g1, dis):
    deg = d0[0, :, :1] + d1[0, :, :1] + 1.0
    dv = lax.rsqrt(deg)
    dis[...] = dv
    g1[...] = x[...] * dv


def _tc_scale(degp, x):
    return pl.pallas_call(
        _tc_scale_body,
        grid=(GRID,),
        in_specs=[_p0(16), _p1(16), _rb(D)],
        out_specs=[_rb(D), _rb(1)],
        out_shape=[
            jax.ShapeDtypeStruct((N, D), jnp.float32),
            jax.ShapeDtypeStruct((N, 1), jnp.float32),
        ],
    )(degp, degp, x)


def _tc_z1_body(s0, s1, g1, dis, W1, b1, z, st):
    p = (s0[0] + s1[0] + g1[...]) * dis[...]
    zv = jnp.dot(p, W1[...], preferred_element_type=jnp.float32) + b1[...]
    z[...] = zv
    cs = jnp.sum(zv, axis=0).reshape(1, 1, -1)
    cq = jnp.sum(zv * zv, axis=0).reshape(1, 1, -1)
    st[...] = jnp.concatenate([cs, cq], axis=1)


def _tc_z1(sp, g1, dis, W1, b1):
    dh = W1.shape[1]
    return pl.pallas_call(
        _tc_z1_body,
        grid=(GRID,),
        in_specs=[_p0(D), _p1(D), _rb(D), _rb(1), _full((D, dh)), _full((1, dh))],
        out_specs=[_rb(dh), pl.BlockSpec((1, 2, dh), lambda i: (i, 0, 0))],
        out_shape=[
            jax.ShapeDtypeStruct((N, dh), jnp.float32),
            jax.ShapeDtypeStruct((GRID, 2, dh), jnp.float32),
        ],
    )(sp, sp, g1, dis, W1, b1)


def _bn_relu(z, st, gamma, beta):
    ssum = jnp.sum(st[...][:, 0, :], axis=0)
    ssq = jnp.sum(st[...][:, 1, :], axis=0)
    mean = ssum * (1.0 / N)
    var = ssq * (1.0 / N) - mean * mean
    inv = lax.rsqrt(var + 1e-5)
    g = gamma[...][0]
    sc = (inv * g).reshape(1, -1)
    sh = (beta[...][0] - mean * inv * g).reshape(1, -1)
    return jnp.maximum(z[...] * sc + sh, 0.0)


def _tc_x1_body(z, st, gamma, beta, dis, W2, Wp, g2, r):
    x1 = _bn_relu(z, st, gamma, beta)
    g2[...] = jnp.dot(x1, W2[...], preferred_element_type=jnp.float32) * dis[...]
    r[...] = jnp.dot(x1, Wp[...], preferred_element_type=jnp.float32)


def _tc_x1(z1, st1, gamma, beta, dis, W2, Wp):
    dh = z1.shape[1]
    return pl.pallas_call(
        _tc_x1_body,
        grid=(GRID,),
        in_specs=[_rb(dh), _full((GRID, 2, dh)), _full((1, dh)), _full((1, dh)),
                  _rb(1), _full((dh, D)), _full((dh, D))],
        out_specs=[_rb(D), _rb(D)],
        out_shape=[
            jax.ShapeDtypeStruct((N, D), jnp.float32),
            jax.ShapeDtypeStruct((N, D), jnp.float32),
        ],
    )(z1, st1, gamma, beta, dis, W2, Wp)


def _tc_z2_body(s0, s1, g2, dis, b2, z, st):
    zv = (s0[0] + s1[0] + g2[...]) * dis[...] + b2[...]
    z[...] = zv
    cs = jnp.sum(zv, axis=0).reshape(1, 1, -1)
    cq = jnp.sum(zv * zv, axis=0).reshape(1, 1, -1)
    st[...] = jnp.concatenate([cs, cq], axis=1)


def _tc_z2(sp, g2, dis, b2):
    return pl.pallas_call(
        _tc_z2_body,
        grid=(GRID,),
        in_specs=[_p0(D), _p1(D), _rb(D), _rb(1), _full((1, D))],
        out_specs=[_rb(D), pl.BlockSpec((1, 2, D), lambda i: (i, 0, 0))],
        out_shape=[
            jax.ShapeDtypeStruct((N, D), jnp.float32),
            jax.ShapeDtypeStruct((GRID, 2, D), jnp.float32),
        ],
    )(sp, sp, g2, dis, b2)


def _tc_x2_body(z, st, gamma, beta, dis, W3, g3):
    x2 = _bn_relu(z, st, gamma, beta)
    g3[...] = jnp.dot(x2, W3[...], preferred_element_type=jnp.float32) * dis[...]


def _tc_x2(z2, st2, gamma, beta, dis, W3):
    return pl.pallas_call(
        _tc_x2_body,
        grid=(GRID,),
        in_specs=[_rb(D), _full((GRID, 2, D)), _full((1, D)), _full((1, D)),
                  _rb(1), _full((D, D))],
        out_specs=_rb(D),
        out_shape=jax.ShapeDtypeStruct((N, D), jnp.float32),
    )(z2, st2, gamma, beta, dis, W3)


def _tc_out_body(s0, s1, g3, dis, b3, r, bp, o):
    o[...] = ((s0[0] + s1[0] + g3[...]) * dis[...] + b3[...] + r[...]
              + bp[...])


def _tc_out(sp, g3, dis, b3, r, bp):
    return pl.pallas_call(
        _tc_out_body,
        grid=(GRID,),
        in_specs=[_p0(D), _p1(D), _rb(D), _rb(1), _full((1, D)), _rb(D),
                  _full((1, D))],
        out_specs=_rb(D),
        out_shape=jax.ShapeDtypeStruct((N, D), jnp.float32),
    )(sp, sp, g3, dis, b3, r, bp)


def kernel(x, edge_index, W1, b1, gamma1, beta1, W2, b2, gamma2, beta2,
           W3, b3, Wproj, bproj):
    pad = EPAD - E
    src = jnp.concatenate([edge_index[0], jnp.zeros((pad,), jnp.int32)])
    pad_dst = N + (jnp.arange(pad, dtype=jnp.int32) % (NACC - N))
    dst = jnp.concatenate([edge_index[1], pad_dst])
    comb = ((src << SHIFT) | dst).reshape(NCHK, B)

    degp = _sc_degree(comb)
    g1, dis = _tc_scale(degp, x)

    s1 = _sc_prop(g1, comb)
    z1, st1 = _tc_z1(s1, g1, dis, W1, b1.reshape(1, -1))
    g2, r = _tc_x1(z1, st1, gamma1.reshape(1, -1), beta1.reshape(1, -1),
                   dis, W2, Wproj)

    s2 = _sc_prop(g2, comb)
    z2, st2 = _tc_z2(s2, g2, dis, b2.reshape(1, -1))
    g3 = _tc_x2(z2, st2, gamma2.reshape(1, -1), beta2.reshape(1, -1), dis, W3)

    s3 = _sc_prop(g3, comb)
    return _tc_out(s3, g3, dis, b3.reshape(1, -1), r, bproj.reshape(1, -1))


# speedup vs baseline: 3.3426x; 3.0212x over previous
import functools

import jax
import jax.numpy as jnp
from jax import lax
from jax.experimental import pallas as pl
from jax.experimental.pallas import tpu as pltpu
from jax.experimental.pallas import tpu_sc as plsc

N = 10000
E = 320000
D = 128

NC, NS = 2, 16
NW = NC * NS
B = 128
SHIFT = 14
CH0 = 80
CH1 = 80
C1CHK = NS * CH1
NCHK = NS * (CH0 + CH1)
EPAD = NCHK * B
DCH = NCHK // NW
NACC = 10240
RPT = NACC // NS
ZR = NACC // NS


def _fill_vmem_2d(ref, rows, width, value):
    val = jnp.full((16,), value, jnp.float32)

    def row(i, _):
        for j in range(width // 16):
            ref[i, pl.ds(j * 16, 16)] = val
        return 0

    lax.fori_loop(0, rows, row, 0)


def _zero_acc_slice(zbuf, acc, s):
    zb = s * ZR
    done = 0
    while done < ZR:
        n = min(B, ZR - done)
        pltpu.sync_copy(zbuf.at[pl.ds(0, n)], acc.at[pl.ds(zb + done, n)])
        done += n


def _unpack_dst(comb, i, dstq):
    for j in range(B // 16):
        v = comb[i, pl.ds(j * 16, 16)]
        dstq[pl.ds(j * 16, 16)] = lax.bitwise_and(v, (1 << SHIFT) - 1)


def _unpack_src(comb, i, srcq):
    for j in range(B // 16):
        v = comb[i, pl.ds(j * 16, 16)]
        srcq[pl.ds(j * 16, 16)] = lax.shift_right_logical(v, SHIFT)


@functools.cache
def _sc_mesh():
    return plsc.VectorSubcoreMesh(core_axis_name="c", subcore_axis_name="s",
                                  num_cores=NC, num_subcores=NS)


@functools.cache
def _sc_degree_kernel():
    return pl.kernel(
        _sc_degree_body,
        out_type=jax.ShapeDtypeStruct((NC, NACC, 16), jnp.float32),
        mesh=_sc_mesh(),
        scratch_types=[
            pltpu.VMEM((DCH, B), jnp.int32),
            pltpu.VMEM((B,), jnp.int32),
            pltpu.VMEM((B, 16), jnp.float32),
            pltpu.VMEM((B, 16), jnp.float32),
            pltpu.VMEM_SHARED((NACC, 16), jnp.float32),
        ],
    )


def _sc_degree(comb):
    return _sc_degree_kernel()(comb)


def _sc_degree_body(comb_hbm, out_hbm, comb, dstq, ones, zbuf, dacc):
    c = lax.axis_index("c")
    s = lax.axis_index("s")
    wid = s * NC + c
    _fill_vmem_2d(ones, B, 16, 1.0)
    _fill_vmem_2d(zbuf, B, 16, 0.0)
    _zero_acc_slice(zbuf, dacc, s)
    pltpu.sync_copy(comb_hbm.at[pl.ds(wid * DCH, DCH)], comb)
    plsc.subcore_barrier()

    def chunk(i, _):
        _unpack_dst(comb, i, dstq)
        pltpu.sync_copy(ones, dacc.at[dstq], add=True)
        return 0

    lax.fori_loop(0, DCH, chunk, 0)
    plsc.subcore_barrier()
    ob = s * RPT
    pltpu.sync_copy(dacc.at[pl.ds(ob, RPT)], out_hbm.at[c, pl.ds(ob, RPT)])


@functools.cache
def _sc_prop_kernel():
    return pl.kernel(
        _sc_prop_body,
        out_type=jax.ShapeDtypeStruct((NC, NACC, D), jnp.float32),
        mesh=_sc_mesh(),
        scratch_types=[
            pltpu.VMEM((CH0, B), jnp.int32),
            pltpu.VMEM((B,), jnp.int32),
            pltpu.VMEM((B,), jnp.int32),
            pltpu.VMEM((B,), jnp.int32),
            pltpu.VMEM((B,), jnp.int32),
            pltpu.VMEM((B, D), jnp.float32),
            pltpu.VMEM((B, D), jnp.float32),
            pltpu.VMEM_SHARED((NACC, D), jnp.float32),
            pltpu.SemaphoreType.DMA,
            pltpu.SemaphoreType.DMA,
        ],
    )


def _sc_prop(g, comb):
    return _sc_prop_kernel()(g, comb)


def _prop_chunks(g_hbm, comb_hbm, comb, srcq0, srcq1, dstq0, dstq1,
                 rows0, rows1, acc, sem0, sem1, base, nch):
    pltpu.sync_copy(comb_hbm.at[pl.ds(base, nch)], comb.at[pl.ds(0, nch)])
    _unpack_src(comb, 0, srcq0)
    _unpack_dst(comb, 0, dstq0)
    pltpu.async_copy(g_hbm.at[srcq0], rows0, sem0)
    _unpack_src(comb, 1, srcq1)
    _unpack_dst(comb, 1, dstq1)

    def pair(k, _):
        i = 2 * k
        pltpu.make_async_copy(g_hbm.at[srcq0], rows0, sem0).wait()
        pltpu.async_copy(g_hbm.at[srcq1], rows1, sem1)
        pltpu.sync_copy(rows0, acc.at[dstq0], add=True)
        _unpack_src(comb, i + 2, srcq0)
        _unpack_dst(comb, i + 2, dstq0)
        pltpu.make_async_copy(g_hbm.at[srcq1], rows1, sem1).wait()
        pltpu.async_copy(g_hbm.at[srcq0], rows0, sem0)
        pltpu.sync_copy(rows1, acc.at[dstq1], add=True)
        _unpack_src(comb, i + 3, srcq1)
        _unpack_dst(comb, i + 3, dstq1)
        return 0

    lax.fori_loop(0, nch // 2 - 1, pair, 0)
    pltpu.make_async_copy(g_hbm.at[srcq0], rows0, sem0).wait()
    pltpu.async_copy(g_hbm.at[srcq1], rows1, sem1)
    pltpu.sync_copy(rows0, acc.at[dstq0], add=True)
    pltpu.make_async_copy(g_hbm.at[srcq1], rows1, sem1).wait()
    pltpu.sync_copy(rows1, acc.at[dstq1], add=True)


def _sc_prop_body(g_hbm, comb_hbm, out_hbm, comb, srcq0, srcq1, dstq0, dstq1,
                  rows0, rows1, acc, sem0, sem1):
    c = lax.axis_index("c")
    s = lax.axis_index("s")
    _fill_vmem_2d(rows0, B, D, 0.0)
    _zero_acc_slice(rows0, acc, s)
    plsc.subcore_barrier()

    args = (g_hbm, comb_hbm, comb, srcq0, srcq1, dstq0, dstq1,
            rows0, rows1, acc, sem0, sem1)

    @pl.when(c == 0)
    def _():
        _prop_chunks(*args, C1CHK + s * CH0, CH0)

    @pl.when(c != 0)
    def _():
        _prop_chunks(*args, s * CH1, CH1)

    plsc.subcore_barrier()
    ob = s * RPT
    pltpu.sync_copy(acc.at[pl.ds(ob, RPT)], out_hbm.at[c, pl.ds(ob, RPT)])


RB = 2000
GRID = N // RB


def _rb(width):
    return pl.BlockSpec((RB, width), lambda i: (i, 0))


def _p0(width):
    return pl.BlockSpec((1, RB, width), lambda i: (0, i, 0))


def _p1(width):
    return pl.BlockSpec((1, RB, width), lambda i: (1, i, 0))


def _full(shape):
    return pl.BlockSpec(shape, lambda i: tuple(0 for _ in shape))


def _tc_scale_body(d0, d1, x, g1, dis):
    deg = d0[0, :, :1] + d1[0, :, :1] + 1.0
    dv = lax.rsqrt(deg)
    dis[...] = dv
    g1[...] = x[...] * dv


def _tc_scale(degp, x):
    return pl.pallas_call(
        _tc_scale_body,
        grid=(GRID,),
        in_specs=[_p0(16), _p1(16), _rb(D)],
        out_specs=[_rb(D), _rb(1)],
        out_shape=[
            jax.ShapeDtypeStruct((N, D), jnp.float32),
            jax.ShapeDtypeStruct((N, 1), jnp.float32),
        ],
    )(degp, degp, x)


def _tc_z1_body(s0, s1, g1, dis, W1, b1, z, st):
    p = (s0[0] + s1[0] + g1[...]) * dis[...]
    zv = jnp.dot(p, W1[...], preferred_element_type=jnp.float32) + b1[...]
    z[...] = zv
    cs = jnp.sum(zv, axis=0).reshape(1, 1, -1)
    cq = jnp.sum(zv * zv, axis=0).reshape(1, 1, -1)
    st[...] = jnp.concatenate([cs, cq], axis=1)


def _tc_z1(sp, g1, dis, W1, b1):
    dh = W1.shape[1]
    return pl.pallas_call(
        _tc_z1_body,
        grid=(GRID,),
        in_specs=[_p0(D), _p1(D), _rb(D), _rb(1), _full((D, dh)), _full((1, dh))],
        out_specs=[_rb(dh), pl.BlockSpec((1, 2, dh), lambda i: (i, 0, 0))],
        out_shape=[
            jax.ShapeDtypeStruct((N, dh), jnp.float32),
            jax.ShapeDtypeStruct((GRID, 2, dh), jnp.float32),
        ],
    )(sp, sp, g1, dis, W1, b1)


def _bn_relu(z, st, gamma, beta):
    ssum = jnp.sum(st[...][:, 0, :], axis=0)
    ssq = jnp.sum(st[...][:, 1, :], axis=0)
    mean = ssum * (1.0 / N)
    var = ssq * (1.0 / N) - mean * mean
    inv = lax.rsqrt(var + 1e-5)
    g = gamma[...][0]
    sc = (inv * g).reshape(1, -1)
    sh = (beta[...][0] - mean * inv * g).reshape(1, -1)
    return jnp.maximum(z[...] * sc + sh, 0.0)


def _tc_x1_body(z, st, gamma, beta, dis, W2, Wp, g2, r):
    x1 = _bn_relu(z, st, gamma, beta)
    g2[...] = jnp.dot(x1, W2[...], preferred_element_type=jnp.float32) * dis[...]
    r[...] = jnp.dot(x1, Wp[...], preferred_element_type=jnp.float32)


def _tc_x1(z1, st1, gamma, beta, dis, W2, Wp):
    dh = z1.shape[1]
    return pl.pallas_call(
        _tc_x1_body,
        grid=(GRID,),
        in_specs=[_rb(dh), _full((GRID, 2, dh)), _full((1, dh)), _full((1, dh)),
                  _rb(1), _full((dh, D)), _full((dh, D))],
        out_specs=[_rb(D), _rb(D)],
        out_shape=[
            jax.ShapeDtypeStruct((N, D), jnp.float32),
            jax.ShapeDtypeStruct((N, D), jnp.float32),
        ],
    )(z1, st1, gamma, beta, dis, W2, Wp)


def _tc_z2_body(s0, s1, g2, dis, b2, z, st):
    zv = (s0[0] + s1[0] + g2[...]) * dis[...] + b2[...]
    z[...] = zv
    cs = jnp.sum(zv, axis=0).reshape(1, 1, -1)
    cq = jnp.sum(zv * zv, axis=0).reshape(1, 1, -1)
    st[...] = jnp.concatenate([cs, cq], axis=1)


def _tc_z2(sp, g2, dis, b2):
    return pl.pallas_call(
        _tc_z2_body,
        grid=(GRID,),
        in_specs=[_p0(D), _p1(D), _rb(D), _rb(1), _full((1, D))],
        out_specs=[_rb(D), pl.BlockSpec((1, 2, D), lambda i: (i, 0, 0))],
        out_shape=[
            jax.ShapeDtypeStruct((N, D), jnp.float32),
            jax.ShapeDtypeStruct((GRID, 2, D), jnp.float32),
        ],
    )(sp, sp, g2, dis, b2)


def _tc_x2_body(z, st, gamma, beta, dis, W3, g3):
    x2 = _bn_relu(z, st, gamma, beta)
    g3[...] = jnp.dot(x2, W3[...], preferred_element_type=jnp.float32) * dis[...]


def _tc_x2(z2, st2, gamma, beta, dis, W3):
    return pl.pallas_call(
        _tc_x2_body,
        grid=(GRID,),
        in_specs=[_rb(D), _full((GRID, 2, D)), _full((1, D)), _full((1, D)),
                  _rb(1), _full((D, D))],
        out_specs=_rb(D),
        out_shape=jax.ShapeDtypeStruct((N, D), jnp.float32),
    )(z2, st2, gamma, beta, dis, W3)


def _tc_out_body(s0, s1, g3, dis, b3, r, bp, o):
    o[...] = ((s0[0] + s1[0] + g3[...]) * dis[...] + b3[...] + r[...]
              + bp[...])


def _tc_out(sp, g3, dis, b3, r, bp):
    return pl.pallas_call(
        _tc_out_body,
        grid=(GRID,),
        in_specs=[_p0(D), _p1(D), _rb(D), _rb(1), _full((1, D)), _rb(D),
                  _full((1, D))],
        out_specs=_rb(D),
        out_shape=jax.ShapeDtypeStruct((N, D), jnp.float32),
    )(sp, sp, g3, dis, b3, r, bp)


def kernel(x, edge_index, W1, b1, gamma1, beta1, W2, b2, gamma2, beta2,
           W3, b3, Wproj, bproj):
    pad = EPAD - E
    pad_src = jnp.arange(pad, dtype=jnp.int32) % N
    pad_dst = N + (jnp.arange(pad, dtype=jnp.int32) % (NACC - N))
    src = jnp.concatenate([edge_index[0], pad_src])
    dst = jnp.concatenate([edge_index[1], pad_dst])
    comb = ((src << SHIFT) | dst).reshape(NCHK, B)

    degp = _sc_degree(comb)
    g1, dis = _tc_scale(degp, x)

    s1 = _sc_prop(g1, comb)
    z1, st1 = _tc_z1(s1, g1, dis, W1, b1.reshape(1, -1))
    g2, r = _tc_x1(z1, st1, gamma1.reshape(1, -1), beta1.reshape(1, -1),
                   dis, W2, Wproj)

    s2 = _sc_prop(g2, comb)
    z2, st2 = _tc_z2(s2, g2, dis, b2.reshape(1, -1))
    g3 = _tc_x2(z2, st2, gamma2.reshape(1, -1), beta2.reshape(1, -1), dis, W3)

    s3 = _sc_prop(g3, comb)
    return _tc_out(s3, g3, dis, b3.reshape(1, -1), r, bproj.reshape(1, -1))
